# baseline (speedup 1.0000x reference)
import jax
import jax.numpy as jnp
from jax.experimental import pallas as pl
from jax.experimental.pallas import tpu as pltpu

_T_BLOCK = 1024
_NUM_EMB = 512


def _ln(x, g, b, eps=1e-5):
    m = jnp.mean(x, axis=-1, keepdims=True)
    v = jnp.mean((x - m) ** 2, axis=-1, keepdims=True)
    return (x - m) / jnp.sqrt(v + eps) * g + b


def _first_argmin(s):
    smin = jnp.min(s, axis=1, keepdims=True)
    iota = jax.lax.broadcasted_iota(jnp.int32, s.shape, 1)
    idx = jnp.min(jnp.where(s == smin, iota, s.shape[1]), axis=1)
    onehot = (iota == idx[:, None]).astype(jnp.float32)
    return smin[:, 0], onehot


def _distances(h, cbT):
    return (jnp.sum(h * h, axis=1, keepdims=True)
            + jnp.sum(cbT * cbT, axis=0)[None, :]) - 2.0 * jnp.dot(h, cbT)


def _fused_kernel(*refs):
    (x_ref, W1_ref, b1_ref, g1_ref, be1_ref, W2_ref, b2_ref, g2_ref, be2_ref,
     cbT0_ref, cb0_ref) = refs[:11]
    r_ref, ma_ref, loss_ref, tab_ref = refs[-4:]
    i = pl.program_id(0)

    @pl.when(i == 0)
    def _():
        h = cb0_ref[...]
        loss = jnp.zeros((_NUM_EMB,), jnp.float32)
        pos = 11
        for _ in range(3):
            W1, b1, g1, be1, W2, b2, g2, be2, cb_ref, cbT_ref = refs[pos:pos + 10]
            pos += 10
            h = _ln(jnp.dot(h, W1[...]) + b1[...], g1[...], be1[...])
            h = jnp.maximum(h, 0.0)
            h = _ln(jnp.dot(h, W2[...]) + b2[...], g2[...], be2[...])
            s = _distances(h, cbT_ref[...])
            _, onehot = _first_argmin(s)
            q = jnp.dot(onehot, cb_ref[...])
            loss = loss + jnp.mean((q - h) ** 2, axis=1)
            h = q
        tab_ref[:, 20:52] = h
        r = h
        for _ in range(4):
            W1, b1, g1, be1, W2, b2, g2, be2 = refs[pos:pos + 8]
            pos += 8
            r = _ln(jnp.dot(r, W1[...]) + b1[...], g1[...], be1[...])
            r = jnp.maximum(r, 0.0)
            r = _ln(jnp.dot(r, W2[...]) + b2[...], g2[...], be2[...])
        tab_ref[:, 0:20] = r
        tab_ref[:, 52:53] = loss[:, None]

    h = _ln(jnp.dot(x_ref[...], W1_ref[...]) + b1_ref[...], g1_ref[...], be1_ref[...])
    h = jnp.maximum(h, 0.0)
    h = _ln(jnp.dot(h, W2_ref[...]) + b2_ref[...], g2_ref[...], be2_ref[...])
    cbT = cbT0_ref[...]
    s = _distances(h, cbT)
    dmin, onehot = _first_argmin(s)
    g = jnp.dot(onehot, tab_ref[...])
    r_ref[...] = g[:, 0:20]
    ma_ref[...] = g[:, 20:52]
    part = (jnp.sum(dmin) * (1.0 / cbT.shape[0]) + jnp.sum(g[:, 52])).reshape(1, 1)

    @pl.when(i == 0)
    def _():
        loss_ref[...] = part

    @pl.when(i != 0)
    def _():
        loss_ref[...] += part


def _row(v):
    return v.reshape(1, -1)


def kernel(x, enc_params, dec_params):
    T, din = x.shape
    p0 = enc_params[0]
    cb0 = p0["codebook"]
    num_emb, dim0 = cb0.shape
    d1 = p0["W1"].shape[1]

    inputs = [x, p0["W1"], _row(p0["b1"]), _row(p0["g1"]), _row(p0["be1"]),
              p0["W2"], _row(p0["b2"]), _row(p0["g2"]), _row(p0["be2"]),
              cb0.T, cb0]
    for p in enc_params[1:]:
        inputs += [p["W1"], _row(p["b1"]), _row(p["g1"]), _row(p["be1"]),
                   p["W2"], _row(p["b2"]), _row(p["g2"]), _row(p["be2"]),
                   p["codebook"], p["codebook"].T]
    for p in dec_params:
        inputs += [p["W1"], _row(p["b1"]), _row(p["g1"]), _row(p["be1"]),
                   p["W2"], _row(p["b2"]), _row(p["g2"]), _row(p["be2"])]

    bt = _T_BLOCK
    grid = (T // bt,)
    full = lambda a: pl.BlockSpec(a.shape, lambda i: (0,) * a.ndim)
    in_specs = [pl.BlockSpec((bt, din), lambda i: (i, 0))] + [full(a) for a in inputs[1:]]
    out_r, out_ma, loss = pl.pallas_call(
        _fused_kernel,
        grid=grid,
        in_specs=in_specs,
        out_specs=[
            pl.BlockSpec((bt, 20), lambda i: (i, 0)),
            pl.BlockSpec((bt, 32), lambda i: (i, 0)),
            pl.BlockSpec((1, 1), lambda i: (0, 0)),
        ],
        out_shape=[
            jax.ShapeDtypeStruct((T, 20), jnp.float32),
            jax.ShapeDtypeStruct((T, 32), jnp.float32),
            jax.ShapeDtypeStruct((1, 1), jnp.float32),
        ],
        scratch_shapes=[pltpu.VMEM((num_emb, 53), jnp.float32)],
    )(*inputs)

    vq_loss = (jnp.float32(1.25) / T) * loss[0, 0]
    return out_r, out_ma, vq_loss


# speedup vs baseline: 1.1521x; 1.1521x over previous
import jax
import jax.numpy as jnp
from jax.experimental import pallas as pl
from jax.experimental.pallas import tpu as pltpu

_T_BLOCK = 4096
_NUM_EMB = 512


def _ln(x, g, b, eps=1e-5):
    m = jnp.mean(x, axis=-1, keepdims=True)
    v = jnp.mean((x - m) ** 2, axis=-1, keepdims=True)
    return (x - m) / jnp.sqrt(v + eps) * g + b


def _first_argmin(s):
    smin = jnp.min(s, axis=1, keepdims=True)
    iota = jax.lax.broadcasted_iota(jnp.int32, s.shape, 1)
    idx = jnp.min(jnp.where(s == smin, iota, s.shape[1]), axis=1)
    onehot = (iota == idx[:, None]).astype(jnp.float32)
    return smin[:, 0], onehot


def _distances(h, cbT):
    return (jnp.sum(h * h, axis=1, keepdims=True)
            + jnp.sum(cbT * cbT, axis=0)[None, :]) - 2.0 * jnp.dot(h, cbT)


def _fused_kernel(*refs):
    (x_ref, W1_ref, b1_ref, g1_ref, be1_ref, W2_ref, b2_ref, g2_ref, be2_ref,
     cbT0_ref, cb0_ref) = refs[:11]
    r_ref, ma_ref, loss_ref, tab_ref = refs[-4:]
    i = pl.program_id(0)

    @pl.when(i == 0)
    def _():
        h = cb0_ref[...]
        loss = jnp.zeros((_NUM_EMB,), jnp.float32)
        pos = 11
        for _ in range(3):
            W1, b1, g1, be1, W2, b2, g2, be2, cb_ref, cbT_ref = refs[pos:pos + 10]
            pos += 10
            h = _ln(jnp.dot(h, W1[...]) + b1[...], g1[...], be1[...])
            h = jnp.maximum(h, 0.0)
            h = _ln(jnp.dot(h, W2[...]) + b2[...], g2[...], be2[...])
            s = _distances(h, cbT_ref[...])
            _, onehot = _first_argmin(s)
            q = jnp.dot(onehot, cb_ref[...])
            loss = loss + jnp.mean((q - h) ** 2, axis=1)
            h = q
        tab_ref[:, 20:52] = h
        r = h
        for _ in range(4):
            W1, b1, g1, be1, W2, b2, g2, be2 = refs[pos:pos + 8]
            pos += 8
            r = _ln(jnp.dot(r, W1[...]) + b1[...], g1[...], be1[...])
            r = jnp.maximum(r, 0.0)
            r = _ln(jnp.dot(r, W2[...]) + b2[...], g2[...], be2[...])
        tab_ref[:, 0:20] = r
        tab_ref[:, 52:53] = loss[:, None]

    h = _ln(jnp.dot(x_ref[...], W1_ref[...]) + b1_ref[...], g1_ref[...], be1_ref[...])
    h = jnp.maximum(h, 0.0)
    h = _ln(jnp.dot(h, W2_ref[...]) + b2_ref[...], g2_ref[...], be2_ref[...])
    cbT = cbT0_ref[...]
    s = _distances(h, cbT)
    dmin, onehot = _first_argmin(s)
    g = jnp.dot(onehot, tab_ref[...])
    r_ref[...] = g[:, 0:20]
    ma_ref[...] = g[:, 20:52]
    part = (jnp.sum(dmin) * (1.0 / cbT.shape[0]) + jnp.sum(g[:, 52])).reshape(1, 1)

    @pl.when(i == 0)
    def _():
        loss_ref[...] = part

    @pl.when(i != 0)
    def _():
        loss_ref[...] += part


def _row(v):
    return v.reshape(1, -1)


def kernel(x, enc_params, dec_params):
    T, din = x.shape
    p0 = enc_params[0]
    cb0 = p0["codebook"]
    num_emb, dim0 = cb0.shape
    d1 = p0["W1"].shape[1]

    inputs = [x, p0["W1"], _row(p0["b1"]), _row(p0["g1"]), _row(p0["be1"]),
              p0["W2"], _row(p0["b2"]), _row(p0["g2"]), _row(p0["be2"]),
              cb0.T, cb0]
    for p in enc_params[1:]:
        inputs += [p["W1"], _row(p["b1"]), _row(p["g1"]), _row(p["be1"]),
                   p["W2"], _row(p["b2"]), _row(p["g2"]), _row(p["be2"]),
                   p["codebook"], p["codebook"].T]
    for p in dec_params:
        inputs += [p["W1"], _row(p["b1"]), _row(p["g1"]), _row(p["be1"]),
                   p["W2"], _row(p["b2"]), _row(p["g2"]), _row(p["be2"])]

    bt = _T_BLOCK
    grid = (T // bt,)
    full = lambda a: pl.BlockSpec(a.shape, lambda i: (0,) * a.ndim)
    in_specs = [pl.BlockSpec((bt, din), lambda i: (i, 0))] + [full(a) for a in inputs[1:]]
    out_r, out_ma, loss = pl.pallas_call(
        _fused_kernel,
        grid=grid,
        in_specs=in_specs,
        out_specs=[
            pl.BlockSpec((bt, 20), lambda i: (i, 0)),
            pl.BlockSpec((bt, 32), lambda i: (i, 0)),
            pl.BlockSpec((1, 1), lambda i: (0, 0)),
        ],
        out_shape=[
            jax.ShapeDtypeStruct((T, 20), jnp.float32),
            jax.ShapeDtypeStruct((T, 32), jnp.float32),
            jax.ShapeDtypeStruct((1, 1), jnp.float32),
        ],
        scratch_shapes=[pltpu.VMEM((num_emb, 53), jnp.float32)],
    )(*inputs)

    vq_loss = (jnp.float32(1.25) / T) * loss[0, 0]
    return out_r, out_ma, vq_loss


# speedup vs baseline: 1.1659x; 1.0120x over previous
import jax
import jax.numpy as jnp
from jax.experimental import pallas as pl
from jax.experimental.pallas import tpu as pltpu

_T_BLOCK = 8192
_NUM_EMB = 512


def _ln(x, g, b, eps=1e-5):
    m = jnp.mean(x, axis=-1, keepdims=True)
    v = jnp.mean((x - m) ** 2, axis=-1, keepdims=True)
    return (x - m) / jnp.sqrt(v + eps) * g + b


def _first_argmin(s):
    smin = jnp.min(s, axis=1, keepdims=True)
    iota = jax.lax.broadcasted_iota(jnp.int32, s.shape, 1)
    idx = jnp.min(jnp.where(s == smin, iota, s.shape[1]), axis=1)
    onehot = (iota == idx[:, None]).astype(jnp.float32)
    return smin[:, 0], onehot


def _distances(h, cbT):
    return (jnp.sum(h * h, axis=1, keepdims=True)
            + jnp.sum(cbT * cbT, axis=0)[None, :]) - 2.0 * jnp.dot(h, cbT)


def _fused_kernel(*refs):
    (x_ref, W1_ref, b1_ref, g1_ref, be1_ref, W2_ref, b2_ref, g2_ref, be2_ref,
     cbT0_ref, cb0_ref) = refs[:11]
    r_ref, ma_ref, loss_ref, tab_ref = refs[-4:]
    i = pl.program_id(0)

    @pl.when(i == 0)
    def _():
        h = cb0_ref[...]
        loss = jnp.zeros((_NUM_EMB,), jnp.float32)
        pos = 11
        for _ in range(3):
            W1, b1, g1, be1, W2, b2, g2, be2, cb_ref, cbT_ref = refs[pos:pos + 10]
            pos += 10
            h = _ln(jnp.dot(h, W1[...]) + b1[...], g1[...], be1[...])
            h = jnp.maximum(h, 0.0)
            h = _ln(jnp.dot(h, W2[...]) + b2[...], g2[...], be2[...])
            s = _distances(h, cbT_ref[...])
            _, onehot = _first_argmin(s)
            q = jnp.dot(onehot, cb_ref[...])
            loss = loss + jnp.mean((q - h) ** 2, axis=1)
            h = q
        tab_ref[:, 20:52] = h
        r = h
        for _ in range(4):
            W1, b1, g1, be1, W2, b2, g2, be2 = refs[pos:pos + 8]
            pos += 8
            r = _ln(jnp.dot(r, W1[...]) + b1[...], g1[...], be1[...])
            r = jnp.maximum(r, 0.0)
            r = _ln(jnp.dot(r, W2[...]) + b2[...], g2[...], be2[...])
        tab_ref[:, 0:20] = r
        tab_ref[:, 52:53] = loss[:, None]

    h = _ln(jnp.dot(x_ref[...], W1_ref[...]) + b1_ref[...], g1_ref[...], be1_ref[...])
    h = jnp.maximum(h, 0.0)
    h = _ln(jnp.dot(h, W2_ref[...]) + b2_ref[...], g2_ref[...], be2_ref[...])
    cbT = cbT0_ref[...]
    s = _distances(h, cbT)
    dmin, onehot = _first_argmin(s)
    g = jnp.dot(onehot, tab_ref[...])
    r_ref[...] = g[:, 0:20]
    ma_ref[...] = g[:, 20:52]
    part = (jnp.sum(dmin) * (1.0 / cbT.shape[0]) + jnp.sum(g[:, 52])).reshape(1, 1)

    @pl.when(i == 0)
    def _():
        loss_ref[...] = part

    @pl.when(i != 0)
    def _():
        loss_ref[...] += part


def _row(v):
    return v.reshape(1, -1)


def kernel(x, enc_params, dec_params):
    T, din = x.shape
    p0 = enc_params[0]
    cb0 = p0["codebook"]
    num_emb, dim0 = cb0.shape
    d1 = p0["W1"].shape[1]

    inputs = [x, p0["W1"], _row(p0["b1"]), _row(p0["g1"]), _row(p0["be1"]),
              p0["W2"], _row(p0["b2"]), _row(p0["g2"]), _row(p0["be2"]),
              cb0.T, cb0]
    for p in enc_params[1:]:
        inputs += [p["W1"], _row(p["b1"]), _row(p["g1"]), _row(p["be1"]),
                   p["W2"], _row(p["b2"]), _row(p["g2"]), _row(p["be2"]),
                   p["codebook"], p["codebook"].T]
    for p in dec_params:
        inputs += [p["W1"], _row(p["b1"]), _row(p["g1"]), _row(p["be1"]),
                   p["W2"], _row(p["b2"]), _row(p["g2"]), _row(p["be2"])]

    bt = _T_BLOCK
    grid = (T // bt,)
    full = lambda a: pl.BlockSpec(a.shape, lambda i: (0,) * a.ndim)
    in_specs = [pl.BlockSpec((bt, din), lambda i: (i, 0))] + [full(a) for a in inputs[1:]]
    out_r, out_ma, loss = pl.pallas_call(
        _fused_kernel,
        grid=grid,
        in_specs=in_specs,
        out_specs=[
            pl.BlockSpec((bt, 20), lambda i: (i, 0)),
            pl.BlockSpec((bt, 32), lambda i: (i, 0)),
            pl.BlockSpec((1, 1), lambda i: (0, 0)),
        ],
        out_shape=[
            jax.ShapeDtypeStruct((T, 20), jnp.float32),
            jax.ShapeDtypeStruct((T, 32), jnp.float32),
            jax.ShapeDtypeStruct((1, 1), jnp.float32),
        ],
        scratch_shapes=[pltpu.VMEM((num_emb, 53), jnp.float32)],
    )(*inputs)

    vq_loss = (jnp.float32(1.25) / T) * loss[0, 0]
    return out_r, out_ma, vq_loss
